# baseline (speedup 1.0000x reference)
import functools

import jax
import jax.numpy as jnp
from jax.experimental import pallas as pl
from jax.experimental.pallas import tpu as pltpu

B, L, C, T = 128, 50, 3, 300
E = 8
K = C * T
D = 768


def _moe_kernel(logits_ref, masks_ref, xf_ref, w_ref, b_ref, out_ref, wm_ref):
    bs = xf_ref.shape[0]
    row0 = pl.program_id(0) * bs
    logits = logits_ref[pl.ds(row0, bs), :]
    mask = (masks_ref[pl.ds(row0, bs), :] == 1).astype(jnp.float32)
    m = jnp.max(logits, axis=1, keepdims=True)
    ex = jnp.exp(logits - m)
    gates = ex / jnp.sum(ex, axis=1, keepdims=True)
    gates = gates * mask
    gates = gates / (jnp.sum(gates, axis=1, keepdims=True) + 1e-9)

    for i in range(bs):
        acc = gates[i, 0] * w_ref[0]
        for e in range(1, E):
            acc = acc + gates[i, e] * w_ref[e]
        wm_ref[...] = acc
        out = jnp.dot(xf_ref[i], wm_ref[...],
                      preferred_element_type=jnp.float32)
        out_ref[i] = (out + _mixed_bias(gates[i], b_ref)).astype(jnp.bfloat16)


def _mixed_bias(g, b_ref):
    acc = g[0] * b_ref[0]
    for e in range(1, E):
        acc = acc + g[e] * b_ref[e]
    return acc[None, :]


@functools.partial(jax.jit, static_argnames=("bs",))
def _run(xf, logits, moe_masks, expert_W, expert_b, bs=4):
    grid = (B // bs,)
    out = pl.pallas_call(
        _moe_kernel,
        grid=grid,
        in_specs=[
            pl.BlockSpec((B, E), lambda i: (0, 0)),
            pl.BlockSpec((B, E), lambda i: (0, 0)),
            pl.BlockSpec((bs, L, K), lambda i: (i, 0, 0)),
            pl.BlockSpec((E, K, D), lambda i: (0, 0, 0)),
            pl.BlockSpec((E, D), lambda i: (0, 0)),
        ],
        out_specs=pl.BlockSpec((bs, L, D), lambda i: (i, 0, 0)),
        out_shape=jax.ShapeDtypeStruct((B, L, D), jnp.bfloat16),
        scratch_shapes=[pltpu.VMEM((K, D), jnp.float32)],
    )(logits, moe_masks, xf, expert_W, expert_b)
    return out


def kernel(cycle_curve_data, logits, moe_masks, expert_W, expert_b):
    xf = cycle_curve_data.reshape(B, L, K)
    out = _run(xf, logits, moe_masks.astype(jnp.int32), expert_W, expert_b)
    return (out, jnp.float32(0.0))


# speedup vs baseline: 1.5085x; 1.5085x over previous
import functools

import jax
import jax.numpy as jnp
from jax.experimental import pallas as pl
from jax.experimental.pallas import tpu as pltpu

B, L, C, T = 128, 50, 3, 300
E = 8
K = C * T
D = 768


def _moe_kernel(logits_ref, masks_ref, xf_ref, w_ref, b_ref, out_ref):
    bs = xf_ref.shape[0]
    row0 = pl.program_id(0) * bs
    logits = logits_ref[pl.ds(row0, bs), :]
    mask = (masks_ref[pl.ds(row0, bs), :] == 1).astype(jnp.float32)
    m = jnp.max(logits, axis=1, keepdims=True)
    ex = jnp.exp(logits - m)
    gates = ex / jnp.sum(ex, axis=1, keepdims=True)
    gates = gates * mask
    gates = gates / (jnp.sum(gates, axis=1, keepdims=True) + 1e-9)

    bias = jnp.dot(gates, b_ref[...], preferred_element_type=jnp.float32)
    gates_bf = gates.astype(jnp.bfloat16)

    for i in range(bs):
        acc = gates_bf[i:i + 1, 0:1] * w_ref[0]
        for e in range(1, E):
            acc = acc + gates_bf[i:i + 1, e:e + 1] * w_ref[e]
        out = jnp.dot(xf_ref[i], acc, preferred_element_type=jnp.float32)
        out_ref[i] = (out + bias[i][None, :]).astype(jnp.bfloat16)


@functools.partial(jax.jit, static_argnames=("bs",))
def _run(xf, logits, moe_masks, expert_W, expert_b, bs=8):
    grid = (B // bs,)
    out = pl.pallas_call(
        _moe_kernel,
        grid=grid,
        in_specs=[
            pl.BlockSpec((B, E), lambda i: (0, 0)),
            pl.BlockSpec((B, E), lambda i: (0, 0)),
            pl.BlockSpec((bs, L, K), lambda i: (i, 0, 0)),
            pl.BlockSpec((E, K, D), lambda i: (0, 0, 0)),
            pl.BlockSpec((E, D), lambda i: (0, 0)),
        ],
        out_specs=pl.BlockSpec((bs, L, D), lambda i: (i, 0, 0)),
        out_shape=jax.ShapeDtypeStruct((B, L, D), jnp.bfloat16),
    )(logits, moe_masks, xf, expert_W, expert_b)
    return out


def kernel(cycle_curve_data, logits, moe_masks, expert_W, expert_b):
    xf = cycle_curve_data.reshape(B, L, K).astype(jnp.bfloat16)
    out = _run(xf, logits, moe_masks.astype(jnp.int32),
               expert_W.astype(jnp.bfloat16), expert_b)
    return (out, jnp.float32(0.0))
